# Initial kernel scaffold; baseline (speedup 1.0000x reference)
import jax
import jax.numpy as jnp
from jax.experimental import pallas as pl


def kernel(x, edge_index, Wl1, Wr1, att1, b1, gamma, beta, rmean, rvar, Wl2, Wr2, att2, b2):
    raise NotImplementedError("write your pallas kernel here")


# speedup vs baseline: 2.3921x; 2.3921x over previous
import dataclasses
import functools

import jax
import jax.numpy as jnp
from jax import lax
from jax.experimental import pallas as pl
from jax.experimental.pallas import tpu as pltpu
from jax.experimental.pallas import tpu_sc as plsc

N = 10000
E = 160000
FIN = 128
H1 = 2
C1 = 128
COUT = 64
NEG = 0.2
EPS = 1e-5

NS = 16
LANES = 16
K = 80
G = K // LANES
SCW = 128
CH = 200

R1 = 3400
P1_STARTS = (0, 3400, 6800)
P1_SIZES = (3400, 3400, 3200)

R2 = 2600
P2_OFFS = (0, 2600)
P2_SIZES = (2600, 2400)

_MESH = plsc.VectorSubcoreMesh(
    core_axis_name="c", subcore_axis_name="s", num_cores=2, num_subcores=NS
)

_SC_PARAMS = pltpu.CompilerParams()
if "needs_layout_passes" in pltpu.CompilerParams.__dataclass_fields__:
    _SC_PARAMS = dataclasses.replace(_SC_PARAMS, needs_layout_passes=False)


def _dot(a, b):
    return jnp.dot(a, b, preferred_element_type=jnp.float32,
                   precision=lax.Precision.HIGHEST)


def _lin1_body(x_ref, wl_ref, wr_ref, xl_ref, xr_ref):
    x = x_ref[...]
    xl_ref[...] = _dot(x, wl_ref[...])
    xr_ref[...] = _dot(x, wr_ref[...])


def _lin1(x, Wl1, Wr1):
    nblk = 10
    blk = N // nblk
    return pl.pallas_call(
        _lin1_body,
        grid=(nblk, H1),
        in_specs=[
            pl.BlockSpec((blk, FIN), lambda i, h: (i, 0)),
            pl.BlockSpec((FIN, C1), lambda i, h: (0, h)),
            pl.BlockSpec((FIN, C1), lambda i, h: (0, h)),
        ],
        out_specs=[
            pl.BlockSpec((blk, C1), lambda i, h: (h * nblk + i, 0)),
            pl.BlockSpec((blk, C1), lambda i, h: (h * nblk + i, 0)),
        ],
        out_shape=[
            jax.ShapeDtypeStruct((H1 * N, C1), jnp.float32),
            jax.ShapeDtypeStruct((H1 * N, C1), jnp.float32),
        ],
    )(x, Wl1, Wr1)


def _zero_bufs(zbuf, wbuf):
    zero = jnp.zeros((LANES,), jnp.float32)
    iota16 = lax.iota(jnp.int32, LANES)

    @pl.loop(0, CH)
    def _(r):
        rowv = jnp.full((LANES,), r, jnp.int32)

        @pl.loop(0, SCW, step=LANES)
        def _(cc):
            plsc.store_scatter(zbuf, [rowv, iota16 + cc], zero)

    @pl.loop(0, K)
    def _(r):
        rowv = jnp.full((LANES,), r, jnp.int32)

        @pl.loop(0, SCW, step=LANES)
        def _(cc):
            plsc.store_scatter(wbuf, [rowv, iota16 + cc], zero)


def _zero_acc(zbuf, acc_sh, s, nch):
    for b in range((nch + NS - 1) // NS):
        idx = s + NS * b

        @pl.when(idx < nch)
        def _():
            pltpu.sync_copy(zbuf, acc_sh.at[pl.ds(idx * CH, CH)])


def _copy_acc(acc_sh, dst_view, s, nch):
    for b in range((nch + NS - 1) // NS):
        idx = s + NS * b

        @pl.when(idx < nch)
        def _():
            pltpu.sync_copy(acc_sh.at[pl.ds(idx * CH, CH)],
                            dst_view.at[pl.ds(idx * CH, CH)])


def _attn_weigh(bufA, bufB, wbuf, att_v, iota16, ncols, boff,
                den_t, dstv, ex_all, e0, excol):
    rows = [iota16 + (j * LANES) for j in range(G)]

    def col_body(ccc, alphas):
        colv = jnp.full((LANES,), ccc, jnp.int32)
        av = plsc.load_gather(att_v, [colv])
        out = []
        for j in range(G):
            a = plsc.load_gather(bufA, [rows[j], colv])
            b = plsc.load_gather(bufB, [rows[j], colv + boff])
            z = a + b
            m = jnp.maximum(z, NEG * z)
            out.append(alphas[j] + av * m)
        return tuple(out)

    zeros = tuple(jnp.zeros((LANES,), jnp.float32) for _ in range(G))
    alphas = lax.fori_loop(0, ncols, col_body, zeros, unroll=2)
    exs = [jnp.exp(alphas[j]) for j in range(G)]
    for j in range(G):
        ex_all[pl.ds(e0 + j * LANES, LANES)] = exs[j]
    if excol:
        excolv = jnp.full((LANES,), ncols, jnp.int32)
        for j in range(G):
            plsc.store_scatter(wbuf, [rows[j], excolv], exs[j])
    if den_t is not None:
        for j in range(G):
            dst16 = dstv[pl.ds(e0 + j * LANES, LANES)]
            plsc.addupdate_scatter(den_t, [dst16], exs[j])

    def col_body2(ccc, carry):
        colv = jnp.full((LANES,), ccc, jnp.int32)
        for j in range(G):
            a = plsc.load_gather(bufA, [rows[j], colv])
            plsc.store_scatter(wbuf, [rows[j], colv], a * exs[j])
        return carry

    lax.fori_loop(0, ncols, col_body2, jnp.int32(0), unroll=2)


def _reweigh(bufA, wbuf, iota16, ncols, ex_all, e0, excol):
    rows = [iota16 + (j * LANES) for j in range(G)]
    exs = [ex_all[pl.ds(e0 + j * LANES, LANES)] for j in range(G)]
    if excol:
        excolv = jnp.full((LANES,), ncols, jnp.int32)
        for j in range(G):
            plsc.store_scatter(wbuf, [rows[j], excolv], exs[j])

    def col_body(ccc, carry):
        colv = jnp.full((LANES,), ccc, jnp.int32)
        for j in range(G):
            a = plsc.load_gather(bufA, [rows[j], colv])
            plsc.store_scatter(wbuf, [rows[j], colv], a * exs[j])
        return carry

    lax.fori_loop(0, ncols, col_body, jnp.int32(0), unroll=2)


def _edge1(xl_flat, xr_flat, src, dst, att1):
    per_tile = E // NS

    @functools.partial(
        pl.kernel,
        out_type=[
            jax.ShapeDtypeStruct((H1, N, SCW), jnp.float32),
            jax.ShapeDtypeStruct((H1 * NS * N,), jnp.float32),
        ],
        mesh=_MESH,
        compiler_params=_SC_PARAMS,
        scratch_types=[
            pltpu.VMEM((C1,), jnp.float32),
            pltpu.VMEM((per_tile,), jnp.int32),
            pltpu.VMEM((per_tile,), jnp.int32),
            pltpu.VMEM((per_tile,), jnp.float32),
            pltpu.VMEM((K,), jnp.int32),
            pltpu.VMEM((K,), jnp.int32),
            pltpu.VMEM((K,), jnp.int32),
            pltpu.VMEM((K, C1), jnp.float32),
            pltpu.VMEM((K, C1), jnp.float32),
            pltpu.VMEM((K, SCW), jnp.float32),
            pltpu.VMEM((CH, SCW), jnp.float32),
            pltpu.VMEM((N,), jnp.float32),
            pltpu.VMEM_SHARED((R1 + 8, SCW), jnp.float32),
        ],
    )
    def k(xl_hbm, xr_hbm, src_hbm, dst_hbm, att_hbm, acc_hbm, den_hbm,
          att_v, src_all, dst_all, ex_all, idxl, idxr, dsc, bufA, bufB,
          wbuf, zbuf, den_t, acc_sh):
        c = lax.axis_index("c")
        s = lax.axis_index("s")
        cN = c * N
        pltpu.sync_copy(att_hbm.at[c], att_v)

        ebase = s * per_tile
        pltpu.sync_copy(src_hbm.at[pl.ds(ebase, per_tile)], src_all)
        pltpu.sync_copy(dst_hbm.at[pl.ds(ebase, per_tile)], dst_all)

        zero = jnp.zeros((LANES,), jnp.float32)

        @pl.loop(0, N, step=LANES)
        def _(i):
            den_t[pl.ds(i, LANES)] = zero

        _zero_bufs(zbuf, wbuf)

        iota16 = lax.iota(jnp.int32, LANES)

        for p, (row0, size) in enumerate(zip(P1_STARTS, P1_SIZES)):
            nch = size // CH
            _zero_acc(zbuf, acc_sh, s, nch)
            plsc.subcore_barrier()

            @pl.loop(0, per_tile, step=K)
            def _(e0, p=p, row0=row0, size=size):
                for j in range(G):
                    sl = pl.ds(e0 + j * LANES, LANES)
                    ov = pl.ds(j * LANES, LANES)
                    sv = src_all[sl]
                    dv = dst_all[sl]
                    idxl[ov] = sv + cN
                    if p == 0:
                        idxr[ov] = dv + cN
                    dl = dv - row0
                    ok = (dl >= 0) & (dl < size)
                    dsc[ov] = jnp.where(ok, dl, R1)
                pltpu.sync_copy(xl_hbm.at[idxl], bufA)
                if p == 0:
                    pltpu.sync_copy(xr_hbm.at[idxr], bufB)
                    _attn_weigh(bufA, bufB, wbuf, att_v, iota16, C1, 0,
                                den_t, dst_all, ex_all, e0, False)
                else:
                    _reweigh(bufA, wbuf, iota16, C1, ex_all, e0, False)
                pltpu.sync_copy(wbuf, acc_sh.at[dsc], add=True)

            plsc.subcore_barrier()
            _copy_acc(acc_sh, acc_hbm.at[c, pl.ds(row0, size)], s, nch)
            plsc.subcore_barrier()

        pltpu.sync_copy(den_t, den_hbm.at[pl.ds((c * NS + s) * N, N)])

    return k(xl_flat, xr_flat, src, dst, att1)


def _edge2(h2, src, dst, att2):
    per_tile = E // NS

    @functools.partial(
        pl.kernel,
        out_type=jax.ShapeDtypeStruct((N, SCW), jnp.float32),
        mesh=_MESH,
        compiler_params=_SC_PARAMS,
        scratch_types=[
            pltpu.VMEM((COUT,), jnp.float32),
            pltpu.VMEM((per_tile,), jnp.int32),
            pltpu.VMEM((per_tile,), jnp.int32),
            pltpu.VMEM((per_tile,), jnp.float32),
            pltpu.VMEM((K,), jnp.int32),
            pltpu.VMEM((K,), jnp.int32),
            pltpu.VMEM((K,), jnp.int32),
            pltpu.VMEM((K, SCW), jnp.float32),
            pltpu.VMEM((K, SCW), jnp.float32),
            pltpu.VMEM((K, SCW), jnp.float32),
            pltpu.VMEM((CH, SCW), jnp.float32),
            pltpu.VMEM_SHARED((R2 + 8, SCW), jnp.float32),
        ],
    )
    def k(h2_hbm, src_hbm, dst_hbm, att_hbm, acc_hbm,
          att_v, src_all, dst_all, ex_all, srcv, dstv, dsc, bufA, bufB,
          wbuf, zbuf, acc_sh):
        c = lax.axis_index("c")
        s = lax.axis_index("s")
        cH = c * (N // 2)
        pltpu.sync_copy(att_hbm.at[0], att_v)

        ebase = s * per_tile
        pltpu.sync_copy(src_hbm.at[pl.ds(ebase, per_tile)], src_all)
        pltpu.sync_copy(dst_hbm.at[pl.ds(ebase, per_tile)], dst_all)

        _zero_bufs(zbuf, wbuf)

        iota16 = lax.iota(jnp.int32, LANES)

        for p, (poff, size) in enumerate(zip(P2_OFFS, P2_SIZES)):
            nch = size // CH
            row0 = cH + poff
            _zero_acc(zbuf, acc_sh, s, nch)
            plsc.subcore_barrier()

            @pl.loop(0, per_tile, step=K)
            def _(e0, p=p, row0=row0, size=size):
                for j in range(G):
                    sl = pl.ds(e0 + j * LANES, LANES)
                    ov = pl.ds(j * LANES, LANES)
                    sv = src_all[sl]
                    dv = dst_all[sl]
                    srcv[ov] = sv
                    if p == 0:
                        dstv[ov] = dv
                    dl = dv - row0
                    ok = (dl >= 0) & (dl < size)
                    dsc[ov] = jnp.where(ok, dl, R2)
                pltpu.sync_copy(h2_hbm.at[srcv], bufA)
                if p == 0:
                    pltpu.sync_copy(h2_hbm.at[dstv], bufB)
                    _attn_weigh(bufA, bufB, wbuf, att_v, iota16, COUT, COUT,
                                None, None, ex_all, e0, True)
                else:
                    _reweigh(bufA, wbuf, iota16, COUT, ex_all, e0, True)
                pltpu.sync_copy(wbuf, acc_sh.at[dsc], add=True)

            plsc.subcore_barrier()
            _copy_acc(acc_sh, acc_hbm.at[pl.ds(row0, size)], s, nch)
            plsc.subcore_barrier()

    return k(h2, src, dst, att2)


def _mid_body(acc_ref, den_ref, b1_ref, g_ref, be_ref, rm_ref, rv_ref,
              wl_ref, wr_ref, h2_ref):
    d = jnp.sum(den_ref[...], axis=2)
    h0 = acc_ref[0] / (d[:, 0:1] + 1e-16)
    h1 = acc_ref[1] / (d[:, 1:2] + 1e-16)
    h = jnp.concatenate([h0, h1], axis=1) + b1_ref[...]
    scale = g_ref[...] / jnp.sqrt(rv_ref[...] + EPS)
    h = (h - rm_ref[...]) * scale + be_ref[...]
    h = jnp.maximum(h, 0.0)
    h2_ref[...] = jnp.concatenate(
        [_dot(h, wl_ref[...]), _dot(h, wr_ref[...])], axis=1)


def _mid(acc1, den1, b1, gamma, beta, rmean, rvar, Wl2, Wr2):
    nblk = 10
    blk = N // nblk
    vec = lambda i: (0, 0)
    return pl.pallas_call(
        _mid_body,
        grid=(nblk,),
        in_specs=[
            pl.BlockSpec((H1, blk, SCW), lambda i: (0, i, 0)),
            pl.BlockSpec((blk, H1, NS), lambda i: (i, 0, 0)),
            pl.BlockSpec((1, H1 * C1), vec),
            pl.BlockSpec((1, H1 * C1), vec),
            pl.BlockSpec((1, H1 * C1), vec),
            pl.BlockSpec((1, H1 * C1), vec),
            pl.BlockSpec((1, H1 * C1), vec),
            pl.BlockSpec((H1 * C1, COUT), vec),
            pl.BlockSpec((H1 * C1, COUT), vec),
        ],
        out_specs=pl.BlockSpec((blk, 2 * COUT), lambda i: (i, 0)),
        out_shape=jax.ShapeDtypeStruct((N, 2 * COUT), jnp.float32),
    )(acc1, den1, b1.reshape(1, -1), gamma.reshape(1, -1), beta.reshape(1, -1),
      rmean.reshape(1, -1), rvar.reshape(1, -1), Wl2, Wr2)


def _final_body(acc_ref, b2_ref, out_ref):
    a = acc_ref[...]
    o = a[:, :COUT] / (a[:, COUT:COUT + 1] + 1e-16) + b2_ref[...]
    m = jnp.max(o, axis=1, keepdims=True)
    lse = m + jnp.log(jnp.sum(jnp.exp(o - m), axis=1, keepdims=True))
    out_ref[...] = o - lse


def _final(acc2, b2):
    nblk = 10
    blk = N // nblk
    return pl.pallas_call(
        _final_body,
        grid=(nblk,),
        in_specs=[
            pl.BlockSpec((blk, SCW), lambda i: (i, 0)),
            pl.BlockSpec((1, COUT), lambda i: (0, 0)),
        ],
        out_specs=pl.BlockSpec((blk, COUT), lambda i: (i, 0)),
        out_shape=jax.ShapeDtypeStruct((N, COUT), jnp.float32),
    )(acc2, b2.reshape(1, -1))


def kernel(x, edge_index, Wl1, Wr1, att1, b1, gamma, beta, rmean, rvar,
           Wl2, Wr2, att2, b2):
    src = edge_index[0]
    dst = edge_index[1]
    xl_flat, xr_flat = _lin1(x, Wl1, Wr1)
    acc1, den1_flat = _edge1(xl_flat, xr_flat, src, dst, att1)
    den1 = den1_flat.reshape(H1, NS, N).transpose(2, 0, 1)
    h2 = _mid(acc1, den1, b1, gamma, beta, rmean, rvar, Wl2, Wr2)
    acc2 = _edge2(h2, src, dst, att2)
    return _final(acc2, b2)
